# baseline (speedup 1.0000x reference)
import jax
import jax.numpy as jnp
from jax import lax
from jax.experimental import pallas as pl
from jax.experimental.pallas import tpu as pltpu

K = 32
QT = 128
W = 256
NS = 32


def _ce(va, ia, vb, ib):
    sw = vb < va
    lo = jnp.minimum(va, vb)
    hi = jnp.maximum(va, vb)
    ilo = jnp.where(sw, ib, ia)
    ihi = jnp.where(sw, ia, ib)
    return lo, ilo, hi, ihi


def _sort4(v, i):
    v = list(v)
    i = list(i)
    for a, b in ((0, 1), (2, 3), (0, 2), (1, 3), (1, 2)):
        v[a], i[a], v[b], i[b] = _ce(v[a], i[a], v[b], i[b])
    return v, i


def _merge4(av, ai, bv, bi):
    mv, mi, dmax = [], [], []
    for r in range(4):
        sw = bv[3 - r] < av[r]
        mv.append(jnp.minimum(av[r], bv[3 - r]))
        mi.append(jnp.where(sw, bi[3 - r], ai[r]))
        dmax.append(jnp.maximum(av[r], bv[3 - r]))
    dmin = jnp.minimum(jnp.minimum(dmax[0], dmax[1]),
                       jnp.minimum(dmax[2], dmax[3]))
    for a, b in ((0, 2), (1, 3), (0, 1), (2, 3)):
        mv[a], mi[a], mv[b], mi[b] = _ce(mv[a], mi[a], mv[b], mi[b])
    return mv, mi, dmin


def _knn_body(nq_ref, xt_ref, out_ref, d_ref):
    q = nq_ref[0]
    p = xt_ref[0]
    n = p.shape[1]

    qc = [q[:, d:d + 1] for d in range(3)]
    sq_q = qc[0] * qc[0] + qc[1] * qc[1] + qc[2] * qc[2]

    lane = lax.broadcasted_iota(jnp.int32, (QT, W), 1)
    inf = jnp.float32(jnp.inf)
    bigi = jnp.int32(n)

    inner = jax.lax.dot_general(
        q.astype(jnp.bfloat16), p.astype(jnp.bfloat16),
        (((1,), (0,)), ((), ())),
        preferred_element_type=jnp.float32)

    vals, idxs = [], []
    for r in range(NS):
        pc = [p[d:d + 1, r * W:(r + 1) * W] for d in range(3)]
        sq_p = pc[0] * pc[0] + pc[1] * pc[1] + pc[2] * pc[2]
        vals.append((sq_q - 2.0 * inner[:, r * W:(r + 1) * W]) + sq_p)
        idxs.append(lane + jnp.int32(r * W))

    queues = []
    for g in range(8):
        v, i = _sort4(vals[4 * g:4 * g + 4], idxs[4 * g:4 * g + 4])
        queues.append((v, i))
    dmins = []
    while len(queues) > 1:
        nxt = []
        for a in range(0, len(queues), 2):
            mv, mi, dmin = _merge4(queues[a][0], queues[a][1],
                                   queues[a + 1][0], queues[a + 1][1])
            dmins.append(dmin)
            nxt.append((mv, mi))
        queues = nxt
    (q1, q2, q3, q4), (i1, i2, i3, i4) = queues[0]
    v5 = dmins[0]
    for dm in dmins[1:]:
        v5 = jnp.minimum(v5, dm)

    half = W // 2
    xv = [x[:, :half] for x in (q1, q2, q3, q4)]
    xv += [x[:, half:] for x in (q4, q3, q2, q1)]
    xi = [x[:, :half] for x in (i1, i2, i3, i4)]
    xi += [x[:, half:] for x in (i4, i3, i2, i1)]
    for a, b in ((0, 4), (1, 5), (2, 6), (3, 7),
                 (0, 2), (1, 3), (4, 6), (5, 7),
                 (0, 1), (2, 3), (4, 5), (6, 7)):
        xv[a], xi[a], xv[b], xi[b] = _ce(xv[a], xi[a], xv[b], xi[b])
    v5 = jnp.minimum(v5[:, :half], v5[:, half:])

    outs = []
    m = None
    for _ in range(K):
        m = jnp.min(xv[0], axis=1, keepdims=True)
        eq = xv[0] == m
        oi = jnp.min(jnp.where(eq, xi[0], bigi), axis=1, keepdims=True)
        win = eq & (xi[0] == oi)
        outs.append(oi)
        for t in range(7):
            xv[t] = jnp.where(win, xv[t + 1], xv[t])
            xi[t] = jnp.where(win, xi[t + 1], xi[t])
        xv[7] = jnp.where(win, inf, xv[7])

    fail = jnp.any(jnp.min(v5, axis=1, keepdims=True) <= m)
    out_good = jnp.concatenate(outs, axis=1)

    @pl.when(jnp.logical_not(fail))
    def _():
        out_ref[0] = out_good

    @pl.when(fail)
    def _():
        iota_n = lax.broadcasted_iota(jnp.int32, (QT, n), 1)
        for r in range(NS):
            d_ref[:, r * W:(r + 1) * W] = vals[r]
        d = d_ref[...]
        fouts = []
        for _ in range(K):
            fm = jnp.min(d, axis=1, keepdims=True)
            fi = jnp.min(jnp.where(d == fm, iota_n, bigi), axis=1,
                         keepdims=True)
            fouts.append(fi)
            d = jnp.where(iota_n == fi, inf, d)
        out_ref[0] = jnp.concatenate(fouts, axis=1)


def kernel(xyz, new_xyz):
    b, n, _ = xyz.shape
    m = new_xyz.shape[1]
    xyz_t = jnp.swapaxes(xyz, 1, 2)
    out = pl.pallas_call(
        _knn_body,
        grid=(b, m // QT),
        in_specs=[
            pl.BlockSpec((1, QT, 3), lambda i, j: (i, j, 0)),
            pl.BlockSpec((1, 3, n), lambda i, j: (i, 0, 0)),
        ],
        out_specs=pl.BlockSpec((1, QT, K), lambda i, j: (i, j, 0)),
        out_shape=jax.ShapeDtypeStruct((b, m, K), jnp.int32),
        scratch_shapes=[pltpu.VMEM((QT, n), jnp.float32)],
    )(new_xyz, xyz_t)
    return out.astype(jnp.int64)


# speedup vs baseline: 1.3514x; 1.3514x over previous
import jax
import jax.numpy as jnp
from jax import lax
from jax.experimental import pallas as pl
from jax.experimental.pallas import tpu as pltpu

K = 32
QT = 256
W = 256
NS = 32


def _ce(va, ia, vb, ib):
    sw = vb < va
    lo = jnp.minimum(va, vb)
    hi = jnp.maximum(va, vb)
    ilo = jnp.where(sw, ib, ia)
    ihi = jnp.where(sw, ia, ib)
    return lo, ilo, hi, ihi


def _sort4(v, i):
    v = list(v)
    i = list(i)
    for a, b in ((0, 1), (2, 3), (0, 2), (1, 3), (1, 2)):
        v[a], i[a], v[b], i[b] = _ce(v[a], i[a], v[b], i[b])
    return v, i


def _merge4(av, ai, bv, bi):
    mv, mi, dmax = [], [], []
    for r in range(4):
        sw = bv[3 - r] < av[r]
        mv.append(jnp.minimum(av[r], bv[3 - r]))
        mi.append(jnp.where(sw, bi[3 - r], ai[r]))
        dmax.append(jnp.maximum(av[r], bv[3 - r]))
    dmin = jnp.minimum(jnp.minimum(dmax[0], dmax[1]),
                       jnp.minimum(dmax[2], dmax[3]))
    for a, b in ((0, 2), (1, 3), (0, 1), (2, 3)):
        mv[a], mi[a], mv[b], mi[b] = _ce(mv[a], mi[a], mv[b], mi[b])
    return mv, mi, dmin


def _knn_body(nq_ref, xt_ref, out_ref):
    q = nq_ref[0]
    p = xt_ref[0]
    n = p.shape[1]

    qc = [q[:, d:d + 1] for d in range(3)]
    sq_q = qc[0] * qc[0] + qc[1] * qc[1] + qc[2] * qc[2]

    lane = lax.broadcasted_iota(jnp.int32, (QT, W), 1)
    inf = jnp.float32(jnp.inf)
    bigi = jnp.int32(n)

    inner = jax.lax.dot_general(
        q.astype(jnp.bfloat16), p.astype(jnp.bfloat16),
        (((1,), (0,)), ((), ())),
        preferred_element_type=jnp.float32)

    vals, idxs = [], []
    for r in range(NS):
        pc = [p[d:d + 1, r * W:(r + 1) * W] for d in range(3)]
        sq_p = pc[0] * pc[0] + pc[1] * pc[1] + pc[2] * pc[2]
        vals.append((sq_q - 2.0 * inner[:, r * W:(r + 1) * W]) + sq_p)
        idxs.append(lane + jnp.int32(r * W))

    queues = []
    for g in range(8):
        v, i = _sort4(vals[4 * g:4 * g + 4], idxs[4 * g:4 * g + 4])
        queues.append((v, i))
    dmins = []
    while len(queues) > 1:
        nxt = []
        for a in range(0, len(queues), 2):
            mv, mi, dmin = _merge4(queues[a][0], queues[a][1],
                                   queues[a + 1][0], queues[a + 1][1])
            dmins.append(dmin)
            nxt.append((mv, mi))
        queues = nxt
    (q1, q2, q3, q4), (i1, i2, i3, i4) = queues[0]
    v5 = dmins[0]
    for dm in dmins[1:]:
        v5 = jnp.minimum(v5, dm)

    half = W // 2
    xv = [x[:, :half] for x in (q1, q2, q3, q4)]
    xv += [x[:, half:] for x in (q4, q3, q2, q1)]
    xi = [x[:, :half] for x in (i1, i2, i3, i4)]
    xi += [x[:, half:] for x in (i4, i3, i2, i1)]
    for a, b in ((0, 4), (1, 5), (2, 6), (3, 7),
                 (0, 2), (1, 3), (4, 6), (5, 7),
                 (0, 1), (2, 3), (4, 5), (6, 7)):
        xv[a], xi[a], xv[b], xi[b] = _ce(xv[a], xi[a], xv[b], xi[b])
    v5 = jnp.minimum(v5[:, :half], v5[:, half:])

    outs = []
    m = None
    for _ in range(K):
        m = jnp.min(xv[0], axis=1, keepdims=True)
        eq = xv[0] == m
        oi = jnp.min(jnp.where(eq, xi[0], bigi), axis=1, keepdims=True)
        win = eq & (xi[0] == oi)
        outs.append(oi)
        for t in range(7):
            xv[t] = jnp.where(win, xv[t + 1], xv[t])
            xi[t] = jnp.where(win, xi[t + 1], xi[t])
        xv[7] = jnp.where(win, inf, xv[7])

    fail = jnp.any(jnp.min(v5, axis=1, keepdims=True) <= m)
    out_good = jnp.concatenate(outs, axis=1)

    @pl.when(jnp.logical_not(fail))
    def _():
        out_ref[0] = out_good

    @pl.when(fail)
    def _():
        iota_n = lax.broadcasted_iota(jnp.int32, (QT, n), 1)
        d = jnp.concatenate(vals, axis=1)
        fouts = []
        for _ in range(K):
            fm = jnp.min(d, axis=1, keepdims=True)
            fi = jnp.min(jnp.where(d == fm, iota_n, bigi), axis=1,
                         keepdims=True)
            fouts.append(fi)
            d = jnp.where(iota_n == fi, inf, d)
        out_ref[0] = jnp.concatenate(fouts, axis=1)


def kernel(xyz, new_xyz):
    b, n, _ = xyz.shape
    m = new_xyz.shape[1]
    xyz_t = jnp.swapaxes(xyz, 1, 2)
    out = pl.pallas_call(
        _knn_body,
        grid=(b, m // QT),
        in_specs=[
            pl.BlockSpec((1, QT, 3), lambda i, j: (i, j, 0)),
            pl.BlockSpec((1, 3, n), lambda i, j: (i, 0, 0)),
        ],
        out_specs=pl.BlockSpec((1, QT, K), lambda i, j: (i, j, 0)),
        out_shape=jax.ShapeDtypeStruct((b, m, K), jnp.int32),
    )(new_xyz, xyz_t)
    return out.astype(jnp.int64)
